# baseline (speedup 1.0000x reference)
import jax
import jax.numpy as jnp
from jax import lax
from jax.experimental import pallas as pl
from jax.experimental.pallas import tpu as pltpu

_R, _C = 8, 2500
_NCLS = 80
_CONF = 0.35
_IOU = 0.6
_MAXDET = 100
_NEGINF = float("-inf")


def _sig(x):
    return 1.0 / (1.0 + jnp.exp(-x))


def _nms_body(bt_ref, pt_ref, out_ref):
    x1 = bt_ref[0]
    y1 = bt_ref[1]
    x2 = bt_ref[2]
    y2 = bt_ref[3]

    def cls_step(k, carry):
        bs, bl = carry
        c = pt_ref[k]
        gt = c > bs
        bs = jnp.where(gt, c, bs)
        bl = jnp.where(gt, k.astype(jnp.float32), bl)
        return bs, bl

    best_logit, labf = lax.fori_loop(
        1, _NCLS, cls_step,
        (pt_ref[0], jnp.zeros((_R, _C), jnp.float32)))
    scores = _sig(best_logit)

    s0 = jnp.where(scores > _CONF, scores, _NEGINF)

    mc = jnp.maximum(jnp.maximum(jnp.max(x1), jnp.max(y1)),
                     jnp.maximum(jnp.max(x2), jnp.max(y2)))
    mc1 = mc + 1.0
    off = labf * mc1
    nx1 = x1 + off
    ny1 = y1 + off
    nx2 = x2 + off
    ny2 = y2 + off
    areas = jnp.maximum(nx2 - nx1, 0.0) * jnp.maximum(ny2 - ny1, 0.0)

    idx = (lax.broadcasted_iota(jnp.int32, (_R, _C), 0) * _C
           + lax.broadcasted_iota(jnp.int32, (_R, _C), 1))
    big = jnp.int32(2 ** 30)
    lane = lax.broadcasted_iota(jnp.int32, (1, 128), 1)

    def nms_step(i, s):
        m = jnp.max(s)
        idxs = jnp.where(s == m, idx, big)
        best = jnp.min(idxs)
        one = idxs == best

        def ext(v):
            return jnp.sum(jnp.where(one, v, 0.0))

        ox1 = ext(x1)
        oy1 = ext(y1)
        ox2 = ext(x2)
        oy2 = ext(y2)
        bl = ext(labf)
        ob = bl * mc1
        bx1 = ox1 + ob
        by1 = oy1 + ob
        bx2 = ox2 + ob
        by2 = oy2 + ob

        a1 = jnp.maximum(bx2 - bx1, 0.0) * jnp.maximum(by2 - by1, 0.0)
        iw = jnp.maximum(jnp.minimum(bx2, nx2) - jnp.maximum(bx1, nx1), 0.0)
        ih = jnp.maximum(jnp.minimum(by2, ny2) - jnp.maximum(by1, ny1), 0.0)
        inter = iw * ih
        iou = inter / (a1 + areas - inter + 1e-7)
        sup = iou > _IOU
        s = jnp.where(sup | one, _NEGINF, s)

        row = jnp.where(lane == 0, ox1,
              jnp.where(lane == 1, oy1,
              jnp.where(lane == 2, ox2,
              jnp.where(lane == 3, oy2,
              jnp.where(lane == 4, m,
              jnp.where(lane == 5, bl, 0.0))))))
        out_ref[pl.ds(i, 1), :] = row
        return s

    lax.fori_loop(0, _MAXDET, nms_step, s0, unroll=2)


def _warp_boxes(boxes, m_inv, width, height):
    x1, y1, x2, y2 = boxes[:, 0], boxes[:, 1], boxes[:, 2], boxes[:, 3]
    xs = jnp.stack([x1, x2, x1, x2], axis=1)
    ys = jnp.stack([y1, y1, y2, y2], axis=1)
    ones = jnp.ones_like(xs)
    pts = jnp.stack([xs, ys, ones], axis=2)
    tp = pts @ m_inv.T
    xw = tp[..., 0] / (tp[..., 2] + 1e-9)
    yw = tp[..., 1] / (tp[..., 2] + 1e-9)
    nx1 = jnp.clip(jnp.min(xw, axis=1), 0.0, width)
    ny1 = jnp.clip(jnp.min(yw, axis=1), 0.0, height)
    nx2 = jnp.clip(jnp.max(xw, axis=1), 0.0, width)
    ny2 = jnp.clip(jnp.max(yw, axis=1), 0.0, height)
    return jnp.stack([nx1, ny1, nx2, ny2], axis=1)


def kernel(preds, input, height, width, warp_matrix):
    del input
    p16 = preds.astype(jnp.float16).astype(jnp.float32)
    cx = jax.nn.sigmoid(p16[:, 0]) * width
    cy = jax.nn.sigmoid(p16[:, 1]) * height
    w = jax.nn.sigmoid(p16[:, 2]) * width * 0.3
    h = jax.nn.sigmoid(p16[:, 3]) * height * 0.3
    boxes = jnp.stack(
        [cx - w / 2.0, cy - h / 2.0, cx + w / 2.0, cy + h / 2.0], axis=1)
    bt = boxes.reshape(_R, _C, 4).transpose(2, 0, 1)
    pt = p16[:, 4:].reshape(_R, _C, _NCLS).transpose(2, 0, 1)
    m_inv = jnp.linalg.inv(warp_matrix)
    dets = pl.pallas_call(
        _nms_body,
        out_shape=jax.ShapeDtypeStruct((_MAXDET, 128), jnp.float32),
        in_specs=[
            pl.BlockSpec(memory_space=pltpu.VMEM),
            pl.BlockSpec(memory_space=pltpu.VMEM),
        ],
        out_specs=pl.BlockSpec(memory_space=pltpu.VMEM),
    )(bt, pt)
    sel_boxes = dets[:, :4]
    sel_scores = dets[:, 4]
    sel_labels = dets[:, 5]
    valid = sel_scores != _NEGINF
    warped = _warp_boxes(sel_boxes, m_inv, width, height)
    out = jnp.concatenate(
        [warped, sel_scores[:, None], sel_labels[:, None]], axis=1)
    return jnp.where(valid[:, None], out, 0.0)
